# baseline (speedup 1.0000x reference)
import jax
import jax.numpy as jnp
import numpy as np
from jax.experimental import pallas as pl
from jax.experimental.pallas import tpu as pltpu

_BINS = 10
_EDGES32 = np.arange(_BINS + 1, dtype=np.float32) / np.float32(_BINS)
_LOGITS = [float(np.log(np.float64(e) / (1.0 - np.float64(e))))
           for e in _EDGES32[1:_BINS]]


def _ghm_body(x_ref, y_ref, out_ref, su_ref, ct_ref):
    step = pl.program_id(0)
    nsteps = pl.num_programs(0)

    @pl.when(step == 0)
    def _init():
        for j in range(_BINS):
            su_ref[j] = 0.0
            ct_ref[j] = 0.0

    x = x_ref[...]
    y = y_ref[...]
    t = jnp.where(y == 0, x, -x)
    e = jnp.exp(-jnp.abs(t))
    pe = jnp.maximum(t, 0.0) + jnp.log1p(e)

    su_ref[0] = su_ref[0] + jnp.sum(pe)
    for j in range(1, _BINS):
        mf = jnp.where(t >= _LOGITS[j - 1], 1.0, 0.0)
        su_ref[j] = su_ref[j] + jnp.sum(mf * pe)
        ct_ref[j] = ct_ref[j] + jnp.sum(mf)

    @pl.when(step == nsteps - 1)
    def _finalize():
        total_n = (jnp.float32(x_ref.shape[0])
                   * jnp.asarray(nsteps, jnp.float32))
        num = jnp.float32(0.0)
        acc = jnp.float32(0.0)
        for b in range(_BINS):
            tb = total_n if b == 0 else ct_ref[b]
            tb1 = jnp.float32(0.0) if b == _BINS - 1 else ct_ref[b + 1]
            ub = su_ref[b]
            ub1 = jnp.float32(0.0) if b == _BINS - 1 else su_ref[b + 1]
            cnt = tb - tb1
            s = ub - ub1
            pos = cnt > 0.0
            num = num + jnp.where(pos, 1.0, 0.0)
            acc = acc + jnp.where(pos, s / jnp.maximum(cnt, 1.0), 0.0)
        out_ref[0, 0] = acc / jnp.maximum(num, 1.0)


def kernel(y_pred, y_true):
    n = y_pred.shape[0]
    grid = 8
    chunk = n // grid
    x2 = y_pred.reshape(n)
    y2 = y_true.reshape(n).astype(jnp.int32)
    out = pl.pallas_call(
        _ghm_body,
        grid=(grid,),
        in_specs=[
            pl.BlockSpec((chunk,), lambda i: (i,)),
            pl.BlockSpec((chunk,), lambda i: (i,)),
        ],
        out_specs=pl.BlockSpec(memory_space=pltpu.SMEM),
        out_shape=jax.ShapeDtypeStruct((1, 1), jnp.float32),
        scratch_shapes=[
            pltpu.SMEM((_BINS,), jnp.float32),
            pltpu.SMEM((_BINS,), jnp.float32),
        ],
    )(x2, y2)
    return out[0, 0]


# speedup vs baseline: 1.2048x; 1.2048x over previous
import jax
import jax.numpy as jnp
import numpy as np
from jax.experimental import pallas as pl
from jax.experimental.pallas import tpu as pltpu

_BINS = 10
_EDGES32 = np.arange(_BINS + 1, dtype=np.float32) / np.float32(_BINS)
_LOGITS = [float(np.log(np.float64(e) / (1.0 - np.float64(e))))
           for e in _EDGES32[1:_BINS]]


def _ghm_body(x_ref, y_ref, out_ref, acc_ref):
    step = pl.program_id(0)
    nsteps = pl.num_programs(0)

    @pl.when(step == 0)
    def _init():
        acc_ref[...] = jnp.zeros_like(acc_ref)

    x = x_ref[...]
    y = y_ref[...]
    t = jnp.where(y == 0, x, -x)
    e = jnp.exp(-jnp.abs(t))
    pe = jnp.maximum(t, 0.0) + jnp.log1p(e)

    acc_ref[0, :] = acc_ref[0, :] + jnp.sum(pe, axis=0)
    for j in range(1, _BINS):
        mf = jnp.where(t >= _LOGITS[j - 1], 1.0, 0.0)
        acc_ref[j, :] = acc_ref[j, :] + jnp.sum(mf * pe, axis=0)
        acc_ref[9 + j, :] = acc_ref[9 + j, :] + jnp.sum(mf, axis=0)

    @pl.when(step == nsteps - 1)
    def _finalize():
        total_n = (jnp.float32(x_ref.shape[0] * x_ref.shape[1])
                   * jnp.asarray(nsteps, jnp.float32))
        u = [jnp.sum(acc_ref[j, :]) for j in range(_BINS)]
        tt = [total_n] + [jnp.sum(acc_ref[9 + j, :]) for j in range(1, _BINS)]
        num = jnp.float32(0.0)
        acc = jnp.float32(0.0)
        for b in range(_BINS):
            tb1 = jnp.float32(0.0) if b == _BINS - 1 else tt[b + 1]
            ub1 = jnp.float32(0.0) if b == _BINS - 1 else u[b + 1]
            cnt = tt[b] - tb1
            s = u[b] - ub1
            pos = cnt > 0.0
            num = num + jnp.where(pos, 1.0, 0.0)
            acc = acc + jnp.where(pos, s / jnp.maximum(cnt, 1.0), 0.0)
        out_ref[0, 0] = acc / jnp.maximum(num, 1.0)


def kernel(y_pred, y_true):
    n = y_pred.shape[0]
    cols = 1024
    rows = n // cols
    grid = 8
    bm = rows // grid
    x2 = y_pred.reshape(rows, cols)
    y2 = y_true.reshape(rows, cols).astype(jnp.int32)
    out = pl.pallas_call(
        _ghm_body,
        grid=(grid,),
        in_specs=[
            pl.BlockSpec((bm, cols), lambda i: (i, 0)),
            pl.BlockSpec((bm, cols), lambda i: (i, 0)),
        ],
        out_specs=pl.BlockSpec(memory_space=pltpu.SMEM),
        out_shape=jax.ShapeDtypeStruct((1, 1), jnp.float32),
        scratch_shapes=[
            pltpu.VMEM((2 * _BINS - 1, cols), jnp.float32),
        ],
    )(x2, y2)
    return out[0, 0]


# speedup vs baseline: 9.3718x; 7.7788x over previous
import jax
import jax.numpy as jnp
import numpy as np
from jax.experimental import pallas as pl
from jax.experimental.pallas import tpu as pltpu

_BINS = 10
_EDGES32 = np.arange(_BINS + 1, dtype=np.float32) / np.float32(_BINS)
_LOGITS = [float(np.log(np.float64(e) / (1.0 - np.float64(e))))
           for e in _EDGES32[1:_BINS]]


def _ghm_body(x_ref, y_ref, out_ref, acc_ref):
    step = pl.program_id(0)
    nsteps = pl.num_programs(0)

    @pl.when(step == 0)
    def _init():
        acc_ref[...] = jnp.zeros_like(acc_ref)

    x = x_ref[...]
    y = y_ref[...]
    t = jnp.where(y == 0, x, -x)
    e = jnp.exp(-jnp.abs(t))
    pe = jnp.maximum(t, 0.0) + jnp.log1p(e)

    acc_ref[0, :] = acc_ref[0, :] + jnp.sum(pe, axis=0)
    for j in range(1, _BINS):
        mf = jnp.where(t >= _LOGITS[j - 1], 1.0, 0.0)
        acc_ref[j, :] = acc_ref[j, :] + jnp.sum(mf * pe, axis=0)
        acc_ref[9 + j, :] = acc_ref[9 + j, :] + jnp.sum(mf, axis=0)

    @pl.when(step == nsteps - 1)
    def _finalize():
        total_n = (jnp.float32(x_ref.shape[0] * x_ref.shape[1])
                   * jnp.asarray(nsteps, jnp.float32))
        u = [jnp.sum(acc_ref[j, :]) for j in range(_BINS)]
        tt = [total_n] + [jnp.sum(acc_ref[9 + j, :]) for j in range(1, _BINS)]
        num = jnp.float32(0.0)
        acc = jnp.float32(0.0)
        for b in range(_BINS):
            tb1 = jnp.float32(0.0) if b == _BINS - 1 else tt[b + 1]
            ub1 = jnp.float32(0.0) if b == _BINS - 1 else u[b + 1]
            cnt = tt[b] - tb1
            s = u[b] - ub1
            pos = cnt > 0.0
            num = num + jnp.where(pos, 1.0, 0.0)
            acc = acc + jnp.where(pos, s / jnp.maximum(cnt, 1.0), 0.0)
        out_ref[0, 0] = acc / jnp.maximum(num, 1.0)


def kernel(y_pred, y_true):
    n = y_pred.shape[0]
    cols = 128
    rows = n // cols
    grid = 8
    bm = rows // grid
    x2 = y_pred.reshape(rows, cols)
    y2 = y_true.reshape(rows, cols).astype(jnp.int32)
    out = pl.pallas_call(
        _ghm_body,
        grid=(grid,),
        in_specs=[
            pl.BlockSpec((bm, cols), lambda i: (i, 0)),
            pl.BlockSpec((bm, cols), lambda i: (i, 0)),
        ],
        out_specs=pl.BlockSpec(memory_space=pltpu.SMEM),
        out_shape=jax.ShapeDtypeStruct((1, 1), jnp.float32),
        scratch_shapes=[
            pltpu.VMEM((2 * _BINS - 1, cols), jnp.float32),
        ],
    )(x2, y2)
    return out[0, 0]
